# baseline (speedup 1.0000x reference)
import jax
import jax.numpy as jnp
from jax import lax
from jax.experimental import pallas as pl
from jax.experimental.pallas import tpu as pltpu
from jax.experimental.pallas import tpu_sc as plsc

_GRID_R, _GRID_C = 28, 28
_CELLS = _GRID_R * _GRID_C
_BOX_NUM = 3
_CLASS_NUM = 80
_F = 5 * _BOX_NUM + _CLASS_NUM
_B = 64
_N = _B * _CELLS

_TC_STEPS = 8
_BLK_B = _B // _TC_STEPS

_NC, _NS, _L = 2, 16, 16
_NW = _NC * _NS
_B_W = _B // _NW
_CELLS_W = _B_W * _CELLS
_GROUPS = _CELLS_W // _L


def _tc_body(p_ref, gt_ref, out_ref, acc_ref):
    g = pl.program_id(0)

    @pl.when(g == 0)
    def _init():
        acc_ref[0] = 0.0
        acc_ref[1] = 0.0

    pos = (gt_ref[..., 0:1] > 0.0).astype(jnp.float32)
    d = p_ref[..., 15:_F] - gt_ref[..., 15:_F]
    acc_ref[0] = acc_ref[0] + jnp.sum(d * d * pos)
    acc_ref[1] = acc_ref[1] + jnp.sum(pos)

    @pl.when(g == _TC_STEPS - 1)
    def _fin():
        out_ref[0] = acc_ref[0]
        out_ref[1] = acc_ref[1]


def _tc_class(p, gt):
    return pl.pallas_call(
        _tc_body,
        grid=(_TC_STEPS,),
        in_specs=[
            pl.BlockSpec((_BLK_B, _GRID_R, _GRID_C, _F),
                         lambda g: (g, 0, 0, 0)),
            pl.BlockSpec((_BLK_B, _GRID_R, _GRID_C, _F),
                         lambda g: (g, 0, 0, 0)),
        ],
        out_specs=pl.BlockSpec(memory_space=pltpu.SMEM),
        out_shape=jax.ShapeDtypeStruct((2,), jnp.float32),
        scratch_shapes=[pltpu.SMEM((2,), jnp.float32)],
    )(p, gt)


def _loc_math(pc, gc, ii, jj):
    c0, c1, c2 = pc[0], pc[5], pc[10]
    best = (
        (c0 >= c1) & (c0 >= c2),
        (c1 > c0) & (c1 >= c2),
        (c2 > c0) & (c2 > c1),
    )
    pos = gc[0] > 0.0

    zero = jnp.zeros_like(c0)
    xy_p = zero
    wh_p = zero
    pc_p = zero
    nc_p = zero
    for k in range(_BOX_NUM):
        ck = (c0, c1, c2)[k]
        m = jnp.where(pos & best[k], 1.0, 0.0)
        px, py, pw, ph = pc[5 * k + 1], pc[5 * k + 2], pc[5 * k + 3], pc[5 * k + 4]
        gx, gy, gw, gh = gc[5 * k + 1], gc[5 * k + 2], gc[5 * k + 3], gc[5 * k + 4]

        dx = px - (gx * float(_GRID_C) - jj)
        dy = py - (gy * float(_GRID_R) - ii)
        dw = pw - gw
        dh = ph - gh
        xy_p = xy_p + m * (dx * dx + dy * dy)
        wh_p = wh_p + m * (dw * dw + dh * dh)

        pxg = (px + jj) / float(_GRID_C)
        pyg = (py + ii) / float(_GRID_R)
        ax1 = pxg - pw * 0.5
        ax2 = pxg + pw * 0.5
        ay1 = pyg - ph * 0.5
        ay2 = pyg + ph * 0.5
        bx1 = gx - gw * 0.5
        bx2 = gx + gw * 0.5
        by1 = gy - gh * 0.5
        by2 = gy + gh * 0.5
        iw = jnp.maximum(jnp.minimum(ax2, bx2) - jnp.maximum(ax1, bx1), 0.0)
        ih = jnp.maximum(jnp.minimum(ay2, by2) - jnp.maximum(ay1, by1), 0.0)
        inter = iw * ih
        area_a = jnp.maximum(pw, 0.0) * jnp.maximum(ph, 0.0)
        area_b = jnp.maximum(gw, 0.0) * jnp.maximum(gh, 0.0)
        iou = inter / (area_a + area_b - inter + 1e-10)

        dc = ck - iou
        pc_p = pc_p + m * dc * dc
        nc_p = nc_p + (1.0 - m) * ck * ck
    return xy_p, wh_p, pc_p, nc_p


def _sc_body(p_hbm, gt_hbm, o_hbm, pv, gv, ov, sem_p, sem_g, sem_o):
    wid = lax.axis_index("s") * _NC + lax.axis_index("c")
    cp = pltpu.async_copy(
        p_hbm.at[pl.ds(wid * _B_W, _B_W), :, :, pl.ds(0, 16)], pv, sem_p)
    cg = pltpu.async_copy(
        gt_hbm.at[pl.ds(wid * _B_W, _B_W), :, :, pl.ds(0, 16)], gv, sem_g)
    cp.wait()
    cg.wait()

    lane = lax.iota(jnp.int32, _L)
    z = jnp.zeros((_L,), jnp.float32)

    def step(g, carry):
        xy_a, wh_a, pc_a, nc_a = carry
        rows = g * _L + lane
        b = (rows >= _CELLS).astype(jnp.int32)
        remf = (rows - b * _CELLS).astype(jnp.float32)
        ii = (remf / float(_GRID_C)).astype(jnp.int32).astype(jnp.float32)
        jj = remf - ii * float(_GRID_C)
        r_idx = ii.astype(jnp.int32)
        c_idx = jj.astype(jnp.int32)
        pc = [plsc.load_gather(pv, [b, r_idx, c_idx, jnp.full((_L,), j, jnp.int32)])
              for j in range(15)]
        gc = [plsc.load_gather(gv, [b, r_idx, c_idx, jnp.full((_L,), j, jnp.int32)])
              for j in range(15)]
        xy_p, wh_p, pc_p, nc_p = _loc_math(pc, gc, ii, jj)
        return xy_a + xy_p, wh_a + wh_p, pc_a + pc_p, nc_a + nc_p

    xy_a, wh_a, pc_a, nc_a = lax.fori_loop(0, _GROUPS, step, (z, z, z, z))
    ov[pl.ds(0, _L)] = xy_a
    ov[pl.ds(_L, _L)] = wh_a
    ov[pl.ds(2 * _L, _L)] = pc_a
    ov[pl.ds(3 * _L, _L)] = nc_a
    pltpu.async_copy(ov, o_hbm.at[wid], sem_o).wait()


def _sc_loc(p, gt):
    mesh = plsc.VectorSubcoreMesh(core_axis_name="c", subcore_axis_name="s")
    f = pl.kernel(
        _sc_body,
        out_type=jax.ShapeDtypeStruct((_NW, 4 * _L), jnp.float32),
        mesh=mesh,
        compiler_params=pltpu.CompilerParams(
            use_tc_tiling_on_sc=False, needs_layout_passes=False),
        scratch_types=[
            pltpu.VMEM((_B_W, _GRID_R, _GRID_C, 16), jnp.float32),
            pltpu.VMEM((_B_W, _GRID_R, _GRID_C, 16), jnp.float32),
            pltpu.VMEM((4 * _L,), jnp.float32),
            pltpu.SemaphoreType.DMA,
            pltpu.SemaphoreType.DMA,
            pltpu.SemaphoreType.DMA,
        ],
    )
    return f(p, gt)


def _combine_body(sc_ref, tc_ref, out_ref):
    xy_sum = jnp.sum(sc_ref[:, 0:_L])
    wh_sum = jnp.sum(sc_ref[:, _L:2 * _L])
    pc_sum = jnp.sum(sc_ref[:, 2 * _L:3 * _L])
    nc_sum = jnp.sum(sc_ref[:, 3 * _L:4 * _L])
    class_sum = tc_ref[0]
    npos = tc_ref[1]
    class_loss = class_sum / jnp.maximum(float(_CLASS_NUM) * npos, 1.0)
    xy_loss = xy_sum / jnp.maximum(2.0 * npos, 1.0)
    wh_loss = wh_sum / jnp.maximum(2.0 * npos, 1.0)
    pos_conf = pc_sum / jnp.maximum(npos, 1.0)
    neg_conf = nc_sum / jnp.maximum(float(_BOX_NUM * _N) - npos, 1.0)
    out_ref[0] = (class_loss + 2.0 * pos_conf + 0.5 * neg_conf
                  + 5.0 * xy_loss + 5.0 * wh_loss)
    out_ref[1] = class_loss
    out_ref[2] = xy_loss
    out_ref[3] = wh_loss
    out_ref[4] = pos_conf
    out_ref[5] = neg_conf


def _combine(sc_out, tc_out):
    return pl.pallas_call(
        _combine_body,
        in_specs=[
            pl.BlockSpec(memory_space=pltpu.VMEM),
            pl.BlockSpec(memory_space=pltpu.SMEM),
        ],
        out_specs=pl.BlockSpec(memory_space=pltpu.SMEM),
        out_shape=jax.ShapeDtypeStruct((6,), jnp.float32),
    )(sc_out, tc_out)


@jax.jit
def _yolo_loss(p, gt):
    sc_out = _sc_loc(p, gt)
    tc_out = _tc_class(p, gt)
    out = _combine(sc_out, tc_out)
    return (out[0], out[1], out[2], out[3], out[4], out[5])


def kernel(p, gt):
    return _yolo_loss(p, gt)


# speedup vs baseline: 1.5473x; 1.5473x over previous
import jax
import jax.numpy as jnp
from jax import lax
from jax.experimental import pallas as pl
from jax.experimental.pallas import tpu as pltpu
from jax.experimental.pallas import tpu_sc as plsc

_GRID_R, _GRID_C = 28, 28
_CELLS = _GRID_R * _GRID_C
_BOX_NUM = 3
_CLASS_NUM = 80
_F = 5 * _BOX_NUM + _CLASS_NUM
_B = 64
_N = _B * _CELLS
_RP = 32

_TC_STEPS = 8
_BLK_B = _B // _TC_STEPS

_NC, _NS, _L = 2, 16, 16
_NW = _NC * _NS
_B_W = _B // _NW
_R_CHUNK = 4
_CELLS_CHUNK = _R_CHUNK * _GRID_C
_GROUPS_CHUNK = _CELLS_CHUNK // _L


def _tc_body(p_ref, gt_ref, loc_ref, out_ref, acc_ref):
    g = pl.program_id(0)

    @pl.when(g == 0)
    def _init():
        acc_ref[0] = 0.0
        acc_ref[1] = 0.0

    p = p_ref[...]
    gt = gt_ref[...]
    loc_ref[:, :, 0:_GRID_C, 0:16] = p[..., 0:16]
    loc_ref[:, :, 0:_GRID_C, 16:32] = gt[..., 0:16]

    pos = (gt[..., 0:1] > 0.0).astype(jnp.float32)
    d = p[..., 15:_F] - gt[..., 15:_F]
    acc_ref[0] = acc_ref[0] + jnp.sum(d * d * pos)
    acc_ref[1] = acc_ref[1] + jnp.sum(pos)

    @pl.when(g == _TC_STEPS - 1)
    def _fin():
        out_ref[0] = acc_ref[0]
        out_ref[1] = acc_ref[1]


def _tc_class(p, gt):
    return pl.pallas_call(
        _tc_body,
        grid=(_TC_STEPS,),
        in_specs=[
            pl.BlockSpec((_BLK_B, _GRID_R, _GRID_C, _F),
                         lambda g: (g, 0, 0, 0)),
            pl.BlockSpec((_BLK_B, _GRID_R, _GRID_C, _F),
                         lambda g: (g, 0, 0, 0)),
        ],
        out_specs=[
            pl.BlockSpec((_BLK_B, _GRID_R, _RP, 128),
                         lambda g: (g, 0, 0, 0)),
            pl.BlockSpec(memory_space=pltpu.SMEM),
        ],
        out_shape=[
            jax.ShapeDtypeStruct((_B, _GRID_R, _RP, 128), jnp.float32),
            jax.ShapeDtypeStruct((2,), jnp.float32),
        ],
        scratch_shapes=[pltpu.SMEM((2,), jnp.float32)],
    )(p, gt)


def _loc_math(pc, gc, ii, jj):
    c0, c1, c2 = pc[0], pc[5], pc[10]
    best = (
        (c0 >= c1) & (c0 >= c2),
        (c1 > c0) & (c1 >= c2),
        (c2 > c0) & (c2 > c1),
    )
    pos = gc[0] > 0.0

    zero = jnp.zeros_like(c0)
    xy_p = zero
    wh_p = zero
    pc_p = zero
    nc_p = zero
    for k in range(_BOX_NUM):
        ck = (c0, c1, c2)[k]
        m = jnp.where(pos & best[k], 1.0, 0.0)
        px, py, pw, ph = pc[5 * k + 1], pc[5 * k + 2], pc[5 * k + 3], pc[5 * k + 4]
        gx, gy, gw, gh = gc[5 * k + 1], gc[5 * k + 2], gc[5 * k + 3], gc[5 * k + 4]

        dx = px - (gx * float(_GRID_C) - jj)
        dy = py - (gy * float(_GRID_R) - ii)
        dw = pw - gw
        dh = ph - gh
        xy_p = xy_p + m * (dx * dx + dy * dy)
        wh_p = wh_p + m * (dw * dw + dh * dh)

        pxg = (px + jj) / float(_GRID_C)
        pyg = (py + ii) / float(_GRID_R)
        ax1 = pxg - pw * 0.5
        ax2 = pxg + pw * 0.5
        ay1 = pyg - ph * 0.5
        ay2 = pyg + ph * 0.5
        bx1 = gx - gw * 0.5
        bx2 = gx + gw * 0.5
        by1 = gy - gh * 0.5
        by2 = gy + gh * 0.5
        iw = jnp.maximum(jnp.minimum(ax2, bx2) - jnp.maximum(ax1, bx1), 0.0)
        ih = jnp.maximum(jnp.minimum(ay2, by2) - jnp.maximum(ay1, by1), 0.0)
        inter = iw * ih
        area_a = jnp.maximum(pw, 0.0) * jnp.maximum(ph, 0.0)
        area_b = jnp.maximum(gw, 0.0) * jnp.maximum(gh, 0.0)
        iou = inter / (area_a + area_b - inter + 1e-10)

        dc = ck - iou
        pc_p = pc_p + m * dc * dc
        nc_p = nc_p + (1.0 - m) * ck * ck
    return xy_p, wh_p, pc_p, nc_p


def _sc_body(loc_hbm, o_hbm, lv, ov, sem_l, sem_o):
    wid = lax.axis_index("s") * _NC + lax.axis_index("c")

    lane = lax.iota(jnp.int32, _L)
    z = jnp.zeros((_L,), jnp.float32)

    def batch_loop(bi, carry0):
        bidx = wid * _B_W + bi

        def chunk_loop(rc, carry1):
            r0 = rc * _R_CHUNK
            pltpu.async_copy(
                loc_hbm.at[bidx, pl.ds(r0, _R_CHUNK), :, pl.ds(0, 2 * _L)],
                lv, sem_l).wait()

            def group_loop(g, carry2):
                xy_a, wh_a, pc_a, nc_a = carry2
                lc = g * _L + lane
                lcf = lc.astype(jnp.float32)
                rlf = (lcf / float(_GRID_C)).astype(jnp.int32).astype(jnp.float32)
                jj = lcf - rlf * float(_GRID_C)
                ii = rlf + jnp.full((_L,), r0, jnp.int32).astype(jnp.float32)
                r_idx = rlf.astype(jnp.int32)
                c_idx = jj.astype(jnp.int32)
                pc = [plsc.load_gather(
                          lv, [r_idx, c_idx, jnp.full((_L,), j, jnp.int32)])
                      for j in range(15)]
                gc = [plsc.load_gather(
                          lv, [r_idx, c_idx, jnp.full((_L,), _L + j, jnp.int32)])
                      for j in range(15)]
                xy_p, wh_p, pc_p, nc_p = _loc_math(pc, gc, ii, jj)
                return (xy_a + xy_p, wh_a + wh_p, pc_a + pc_p, nc_a + nc_p)

            return lax.fori_loop(0, _GROUPS_CHUNK, group_loop, carry1)

        return lax.fori_loop(0, _GRID_R // _R_CHUNK, chunk_loop, carry0)

    xy_a, wh_a, pc_a, nc_a = lax.fori_loop(0, _B_W, batch_loop, (z, z, z, z))
    ov[0, pl.ds(0, _L)] = xy_a
    ov[0, pl.ds(_L, _L)] = wh_a
    ov[0, pl.ds(2 * _L, _L)] = pc_a
    ov[0, pl.ds(3 * _L, _L)] = nc_a
    pltpu.async_copy(ov, o_hbm.at[wid], sem_o).wait()


def _sc_loc(loc):
    mesh = plsc.VectorSubcoreMesh(core_axis_name="c", subcore_axis_name="s")
    f = pl.kernel(
        _sc_body,
        out_type=jax.ShapeDtypeStruct((_NW, 1, 4 * _L), jnp.float32),
        mesh=mesh,
        compiler_params=pltpu.CompilerParams(
            use_tc_tiling_on_sc=False, needs_layout_passes=False),
        scratch_types=[
            pltpu.VMEM((_R_CHUNK, _RP, 2 * _L), jnp.float32),
            pltpu.VMEM((1, 4 * _L), jnp.float32),
            pltpu.SemaphoreType.DMA,
            pltpu.SemaphoreType.DMA,
        ],
    )
    return f(loc)


def _combine_body(sc_ref, tc_ref, out_ref):
    xy_sum = jnp.sum(sc_ref[:, 0, 0:_L])
    wh_sum = jnp.sum(sc_ref[:, 0, _L:2 * _L])
    pc_sum = jnp.sum(sc_ref[:, 0, 2 * _L:3 * _L])
    nc_sum = jnp.sum(sc_ref[:, 0, 3 * _L:4 * _L])
    class_sum = tc_ref[0]
    npos = tc_ref[1]
    class_loss = class_sum / jnp.maximum(float(_CLASS_NUM) * npos, 1.0)
    xy_loss = xy_sum / jnp.maximum(2.0 * npos, 1.0)
    wh_loss = wh_sum / jnp.maximum(2.0 * npos, 1.0)
    pos_conf = pc_sum / jnp.maximum(npos, 1.0)
    neg_conf = nc_sum / jnp.maximum(float(_BOX_NUM * _N) - npos, 1.0)
    out_ref[0] = (class_loss + 2.0 * pos_conf + 0.5 * neg_conf
                  + 5.0 * xy_loss + 5.0 * wh_loss)
    out_ref[1] = class_loss
    out_ref[2] = xy_loss
    out_ref[3] = wh_loss
    out_ref[4] = pos_conf
    out_ref[5] = neg_conf


def _combine(sc_out, tc_out):
    return pl.pallas_call(
        _combine_body,
        in_specs=[
            pl.BlockSpec(memory_space=pltpu.VMEM),
            pl.BlockSpec(memory_space=pltpu.SMEM),
        ],
        out_specs=pl.BlockSpec(memory_space=pltpu.SMEM),
        out_shape=jax.ShapeDtypeStruct((6,), jnp.float32),
    )(sc_out, tc_out)


@jax.jit
def _yolo_loss(p, gt):
    loc, tc_out = _tc_class(p, gt)
    sc_out = _sc_loc(loc)
    out = _combine(sc_out, tc_out)
    return (out[0], out[1], out[2], out[3], out[4], out[5])


def kernel(p, gt):
    return _yolo_loss(p, gt)


# speedup vs baseline: 1.6032x; 1.0362x over previous
import jax
import jax.numpy as jnp
from jax import lax
from jax.experimental import pallas as pl
from jax.experimental.pallas import tpu as pltpu
from jax.experimental.pallas import tpu_sc as plsc

_GRID_R, _GRID_C = 28, 28
_CELLS = _GRID_R * _GRID_C
_BOX_NUM = 3
_CLASS_NUM = 80
_F = 5 * _BOX_NUM + _CLASS_NUM
_B = 64
_N = _B * _CELLS
_RP = 32

_TC_STEPS = 8
_BLK_B = _B // _TC_STEPS

_NC, _NS, _L = 2, 16, 16
_NW = _NC * _NS
_B_W = _B // _NW
_R_CHUNK = 4
_CELLS_CHUNK = _R_CHUNK * _GRID_C
_GROUPS_CHUNK = _CELLS_CHUNK // _L


def _tc_body(p_ref, gt_ref, loc_ref, out_ref, acc_ref):
    g = pl.program_id(0)

    @pl.when(g == 0)
    def _init():
        acc_ref[0] = 0.0
        acc_ref[1] = 0.0

    p = p_ref[...]
    gt = gt_ref[...]
    loc_ref[:, :, 0:_GRID_C, 0:16] = p[..., 0:16]
    loc_ref[:, :, 0:_GRID_C, 16:32] = gt[..., 0:16]

    pos = (gt[..., 0:1] > 0.0).astype(jnp.float32)
    d = p[..., 15:_F] - gt[..., 15:_F]
    acc_ref[0] = acc_ref[0] + jnp.sum(d * d * pos)
    acc_ref[1] = acc_ref[1] + jnp.sum(pos)

    @pl.when(g == _TC_STEPS - 1)
    def _fin():
        out_ref[0] = acc_ref[0]
        out_ref[1] = acc_ref[1]


def _tc_class(p, gt):
    return pl.pallas_call(
        _tc_body,
        grid=(_TC_STEPS,),
        in_specs=[
            pl.BlockSpec((_BLK_B, _GRID_R, _GRID_C, _F),
                         lambda g: (g, 0, 0, 0)),
            pl.BlockSpec((_BLK_B, _GRID_R, _GRID_C, _F),
                         lambda g: (g, 0, 0, 0)),
        ],
        out_specs=[
            pl.BlockSpec((_BLK_B, _GRID_R, _RP, 128),
                         lambda g: (g, 0, 0, 0)),
            pl.BlockSpec(memory_space=pltpu.SMEM),
        ],
        out_shape=[
            jax.ShapeDtypeStruct((_B, _GRID_R, _RP, 128), jnp.float32),
            jax.ShapeDtypeStruct((2,), jnp.float32),
        ],
        scratch_shapes=[pltpu.SMEM((2,), jnp.float32)],
    )(p, gt)


def _loc_math(pc, gc, ii, jj):
    c0, c1, c2 = pc[0], pc[5], pc[10]
    best = (
        (c0 >= c1) & (c0 >= c2),
        (c1 > c0) & (c1 >= c2),
        (c2 > c0) & (c2 > c1),
    )
    pos = gc[0] > 0.0

    zero = jnp.zeros_like(c0)
    xy_p = zero
    wh_p = zero
    pc_p = zero
    nc_p = zero
    for k in range(_BOX_NUM):
        ck = (c0, c1, c2)[k]
        m = jnp.where(pos & best[k], 1.0, 0.0)
        px, py, pw, ph = pc[5 * k + 1], pc[5 * k + 2], pc[5 * k + 3], pc[5 * k + 4]
        gx, gy, gw, gh = gc[5 * k + 1], gc[5 * k + 2], gc[5 * k + 3], gc[5 * k + 4]

        dx = px - (gx * float(_GRID_C) - jj)
        dy = py - (gy * float(_GRID_R) - ii)
        dw = pw - gw
        dh = ph - gh
        xy_p = xy_p + m * (dx * dx + dy * dy)
        wh_p = wh_p + m * (dw * dw + dh * dh)

        pxg = (px + jj) / float(_GRID_C)
        pyg = (py + ii) / float(_GRID_R)
        ax1 = pxg - pw * 0.5
        ax2 = pxg + pw * 0.5
        ay1 = pyg - ph * 0.5
        ay2 = pyg + ph * 0.5
        bx1 = gx - gw * 0.5
        bx2 = gx + gw * 0.5
        by1 = gy - gh * 0.5
        by2 = gy + gh * 0.5
        iw = jnp.maximum(jnp.minimum(ax2, bx2) - jnp.maximum(ax1, bx1), 0.0)
        ih = jnp.maximum(jnp.minimum(ay2, by2) - jnp.maximum(ay1, by1), 0.0)
        inter = iw * ih
        area_a = jnp.maximum(pw, 0.0) * jnp.maximum(ph, 0.0)
        area_b = jnp.maximum(gw, 0.0) * jnp.maximum(gh, 0.0)
        iou = inter / (area_a + area_b - inter + 1e-10)

        dc = ck - iou
        pc_p = pc_p + m * dc * dc
        nc_p = nc_p + (1.0 - m) * ck * ck
    return xy_p, wh_p, pc_p, nc_p


def _sc_body(loc_hbm, o_hbm, lv0, lv1, ov, sem0, sem1, sem_o):
    wid = lax.axis_index("s") * _NC + lax.axis_index("c")

    lane = lax.iota(jnp.int32, _L)
    z = jnp.zeros((_L,), jnp.float32)
    bufs = (lv0, lv1)
    sems = (sem0, sem1)
    n_chunks = _B_W * (_GRID_R // _R_CHUNK)

    def issue(ci):
        bi, rc = divmod(ci, _GRID_R // _R_CHUNK)
        return pltpu.async_copy(
            loc_hbm.at[wid * _B_W + bi,
                       pl.ds(rc * _R_CHUNK, _R_CHUNK), :, :],
            bufs[ci % 2], sems[ci % 2])

    def compute(ci, buf, carry):
        rc = ci % (_GRID_R // _R_CHUNK)

        def group_loop(g, carry2):
            xy_a, wh_a, pc_a, nc_a = carry2
            lc = g * _L + lane
            lcf = lc.astype(jnp.float32)
            rlf = (lcf / float(_GRID_C)).astype(jnp.int32).astype(jnp.float32)
            jj = lcf - rlf * float(_GRID_C)
            ii = rlf + float(rc * _R_CHUNK)
            r_idx = rlf.astype(jnp.int32)
            c_idx = jj.astype(jnp.int32)
            pc = [plsc.load_gather(
                      buf, [r_idx, c_idx, jnp.full((_L,), j, jnp.int32)])
                  for j in range(15)]
            gc = [plsc.load_gather(
                      buf, [r_idx, c_idx, jnp.full((_L,), _L + j, jnp.int32)])
                  for j in range(15)]
            xy_p, wh_p, pc_p, nc_p = _loc_math(pc, gc, ii, jj)
            return (xy_a + xy_p, wh_a + wh_p, pc_a + pc_p, nc_a + nc_p)

        return lax.fori_loop(0, _GROUPS_CHUNK, group_loop, carry)

    acc = (z, z, z, z)
    cps = {0: issue(0)}
    for ci in range(n_chunks):
        if ci + 1 < n_chunks:
            cps[ci + 1] = issue(ci + 1)
        cps[ci].wait()
        acc = compute(ci, bufs[ci % 2], acc)

    xy_a, wh_a, pc_a, nc_a = acc
    ov[0, pl.ds(0, _L)] = xy_a
    ov[0, pl.ds(_L, _L)] = wh_a
    ov[0, pl.ds(2 * _L, _L)] = pc_a
    ov[0, pl.ds(3 * _L, _L)] = nc_a
    pltpu.async_copy(ov, o_hbm.at[wid], sem_o).wait()


def _sc_loc(loc):
    mesh = plsc.VectorSubcoreMesh(core_axis_name="c", subcore_axis_name="s")
    f = pl.kernel(
        _sc_body,
        out_type=jax.ShapeDtypeStruct((_NW, 1, 4 * _L), jnp.float32),
        mesh=mesh,
        compiler_params=pltpu.CompilerParams(needs_layout_passes=False),
        scratch_types=[
            pltpu.VMEM((_R_CHUNK, _RP, 128), jnp.float32),
            pltpu.VMEM((_R_CHUNK, _RP, 128), jnp.float32),
            pltpu.VMEM((1, 4 * _L), jnp.float32),
            pltpu.SemaphoreType.DMA,
            pltpu.SemaphoreType.DMA,
            pltpu.SemaphoreType.DMA,
        ],
    )
    return f(loc)


def _combine_body(sc_ref, tc_ref, out_ref):
    xy_sum = jnp.sum(sc_ref[:, 0, 0:_L])
    wh_sum = jnp.sum(sc_ref[:, 0, _L:2 * _L])
    pc_sum = jnp.sum(sc_ref[:, 0, 2 * _L:3 * _L])
    nc_sum = jnp.sum(sc_ref[:, 0, 3 * _L:4 * _L])
    class_sum = tc_ref[0]
    npos = tc_ref[1]
    class_loss = class_sum / jnp.maximum(float(_CLASS_NUM) * npos, 1.0)
    xy_loss = xy_sum / jnp.maximum(2.0 * npos, 1.0)
    wh_loss = wh_sum / jnp.maximum(2.0 * npos, 1.0)
    pos_conf = pc_sum / jnp.maximum(npos, 1.0)
    neg_conf = nc_sum / jnp.maximum(float(_BOX_NUM * _N) - npos, 1.0)
    out_ref[0] = (class_loss + 2.0 * pos_conf + 0.5 * neg_conf
                  + 5.0 * xy_loss + 5.0 * wh_loss)
    out_ref[1] = class_loss
    out_ref[2] = xy_loss
    out_ref[3] = wh_loss
    out_ref[4] = pos_conf
    out_ref[5] = neg_conf


def _combine(sc_out, tc_out):
    return pl.pallas_call(
        _combine_body,
        in_specs=[
            pl.BlockSpec(memory_space=pltpu.VMEM),
            pl.BlockSpec(memory_space=pltpu.SMEM),
        ],
        out_specs=pl.BlockSpec(memory_space=pltpu.SMEM),
        out_shape=jax.ShapeDtypeStruct((6,), jnp.float32),
    )(sc_out, tc_out)


@jax.jit
def _yolo_loss(p, gt):
    loc, tc_out = _tc_class(p, gt)
    sc_out = _sc_loc(loc)
    out = _combine(sc_out, tc_out)
    return (out[0], out[1], out[2], out[3], out[4], out[5])


def kernel(p, gt):
    return _yolo_loss(p, gt)


# speedup vs baseline: 1.6035x; 1.0002x over previous
import jax
import jax.numpy as jnp
from jax import lax
from jax.experimental import pallas as pl
from jax.experimental.pallas import tpu as pltpu
from jax.experimental.pallas import tpu_sc as plsc

_GRID_R, _GRID_C = 28, 28
_CELLS = _GRID_R * _GRID_C
_BOX_NUM = 3
_CLASS_NUM = 80
_F = 5 * _BOX_NUM + _CLASS_NUM
_B = 64
_N = _B * _CELLS
_RP = 32

_TC_STEPS = 8
_BLK_B = _B // _TC_STEPS

_NC, _NS, _L = 2, 16, 16
_NW = _NC * _NS
_B_W = _B // _NW
_R_CHUNK = 4
_CELLS_CHUNK = _R_CHUNK * _GRID_C
_GROUPS_CHUNK = _CELLS_CHUNK // _L


def _tc_body(p_ref, gt_ref, loc_ref, out_ref, acc_ref):
    g = pl.program_id(0)

    @pl.when(g == 0)
    def _init():
        acc_ref[0] = 0.0
        acc_ref[1] = 0.0

    p = p_ref[...]
    gt = gt_ref[...]
    loc_ref[:, :, 0:_GRID_C, 0:16] = p[..., 0:16]
    loc_ref[:, :, 0:_GRID_C, 16:32] = gt[..., 0:16]

    pos = (gt[..., 0:1] > 0.0).astype(jnp.float32)
    d = p[..., 15:_F] - gt[..., 15:_F]
    acc_ref[0] = acc_ref[0] + jnp.sum(d * d * pos)
    acc_ref[1] = acc_ref[1] + jnp.sum(pos)

    @pl.when(g == _TC_STEPS - 1)
    def _fin():
        out_ref[0] = acc_ref[0]
        out_ref[1] = acc_ref[1]


def _tc_class(p, gt):
    return pl.pallas_call(
        _tc_body,
        grid=(_TC_STEPS,),
        in_specs=[
            pl.BlockSpec((_BLK_B, _GRID_R, _GRID_C, _F),
                         lambda g: (g, 0, 0, 0)),
            pl.BlockSpec((_BLK_B, _GRID_R, _GRID_C, _F),
                         lambda g: (g, 0, 0, 0)),
        ],
        out_specs=[
            pl.BlockSpec((_BLK_B, _GRID_R, _RP, 128),
                         lambda g: (g, 0, 0, 0)),
            pl.BlockSpec(memory_space=pltpu.SMEM),
        ],
        out_shape=[
            jax.ShapeDtypeStruct((_B, _GRID_R, _RP, 128), jnp.float32),
            jax.ShapeDtypeStruct((2,), jnp.float32),
        ],
        scratch_shapes=[pltpu.SMEM((2,), jnp.float32)],
        compiler_params=pltpu.CompilerParams(
            allow_input_fusion=[True, True]),
    )(p, gt)


def _loc_math(pc, gc, ii, jj):
    c0, c1, c2 = pc[0], pc[5], pc[10]
    best = (
        (c0 >= c1) & (c0 >= c2),
        (c1 > c0) & (c1 >= c2),
        (c2 > c0) & (c2 > c1),
    )
    pos = gc[0] > 0.0

    zero = jnp.zeros_like(c0)
    xy_p = zero
    wh_p = zero
    pc_p = zero
    nc_p = zero
    for k in range(_BOX_NUM):
        ck = (c0, c1, c2)[k]
        m = jnp.where(pos & best[k], 1.0, 0.0)
        px, py, pw, ph = pc[5 * k + 1], pc[5 * k + 2], pc[5 * k + 3], pc[5 * k + 4]
        gx, gy, gw, gh = gc[5 * k + 1], gc[5 * k + 2], gc[5 * k + 3], gc[5 * k + 4]

        dx = px - (gx * float(_GRID_C) - jj)
        dy = py - (gy * float(_GRID_R) - ii)
        dw = pw - gw
        dh = ph - gh
        xy_p = xy_p + m * (dx * dx + dy * dy)
        wh_p = wh_p + m * (dw * dw + dh * dh)

        pxg = (px + jj) / float(_GRID_C)
        pyg = (py + ii) / float(_GRID_R)
        ax1 = pxg - pw * 0.5
        ax2 = pxg + pw * 0.5
        ay1 = pyg - ph * 0.5
        ay2 = pyg + ph * 0.5
        bx1 = gx - gw * 0.5
        bx2 = gx + gw * 0.5
        by1 = gy - gh * 0.5
        by2 = gy + gh * 0.5
        iw = jnp.maximum(jnp.minimum(ax2, bx2) - jnp.maximum(ax1, bx1), 0.0)
        ih = jnp.maximum(jnp.minimum(ay2, by2) - jnp.maximum(ay1, by1), 0.0)
        inter = iw * ih
        area_a = jnp.maximum(pw, 0.0) * jnp.maximum(ph, 0.0)
        area_b = jnp.maximum(gw, 0.0) * jnp.maximum(gh, 0.0)
        iou = inter / (area_a + area_b - inter + 1e-10)

        dc = ck - iou
        pc_p = pc_p + m * dc * dc
        nc_p = nc_p + (1.0 - m) * ck * ck
    return xy_p, wh_p, pc_p, nc_p


def _sc_body(loc_hbm, o_hbm, lv0, lv1, ov, sem0, sem1, sem_o):
    wid = lax.axis_index("s") * _NC + lax.axis_index("c")

    lane = lax.iota(jnp.int32, _L)
    z = jnp.zeros((_L,), jnp.float32)
    bufs = (lv0, lv1)
    sems = (sem0, sem1)
    n_chunks = _B_W * (_GRID_R // _R_CHUNK)

    def issue(ci):
        bi, rc = divmod(ci, _GRID_R // _R_CHUNK)
        return pltpu.async_copy(
            loc_hbm.at[wid * _B_W + bi,
                       pl.ds(rc * _R_CHUNK, _R_CHUNK), :, :],
            bufs[ci % 2], sems[ci % 2])

    def compute(ci, buf, carry):
        rc = ci % (_GRID_R // _R_CHUNK)

        def group_loop(g, carry2):
            xy_a, wh_a, pc_a, nc_a = carry2
            lc = g * _L + lane
            lcf = lc.astype(jnp.float32)
            rlf = (lcf / float(_GRID_C)).astype(jnp.int32).astype(jnp.float32)
            jj = lcf - rlf * float(_GRID_C)
            ii = rlf + float(rc * _R_CHUNK)
            r_idx = rlf.astype(jnp.int32)
            c_idx = jj.astype(jnp.int32)
            pc = [plsc.load_gather(
                      buf, [r_idx, c_idx, jnp.full((_L,), j, jnp.int32)])
                  for j in range(15)]
            gc = [plsc.load_gather(
                      buf, [r_idx, c_idx, jnp.full((_L,), _L + j, jnp.int32)])
                  for j in range(15)]
            xy_p, wh_p, pc_p, nc_p = _loc_math(pc, gc, ii, jj)
            return (xy_a + xy_p, wh_a + wh_p, pc_a + pc_p, nc_a + nc_p)

        return lax.fori_loop(0, _GROUPS_CHUNK, group_loop, carry)

    acc = (z, z, z, z)
    cps = {0: issue(0)}
    for ci in range(n_chunks):
        if ci + 1 < n_chunks:
            cps[ci + 1] = issue(ci + 1)
        cps[ci].wait()
        acc = compute(ci, bufs[ci % 2], acc)

    xy_a, wh_a, pc_a, nc_a = acc
    ov[0, pl.ds(0, _L)] = xy_a
    ov[0, pl.ds(_L, _L)] = wh_a
    ov[0, pl.ds(2 * _L, _L)] = pc_a
    ov[0, pl.ds(3 * _L, _L)] = nc_a
    pltpu.async_copy(ov, o_hbm.at[wid], sem_o).wait()


def _sc_loc(loc):
    mesh = plsc.VectorSubcoreMesh(core_axis_name="c", subcore_axis_name="s")
    f = pl.kernel(
        _sc_body,
        out_type=jax.ShapeDtypeStruct((_NW, 1, 4 * _L), jnp.float32),
        mesh=mesh,
        compiler_params=pltpu.CompilerParams(needs_layout_passes=False),
        scratch_types=[
            pltpu.VMEM((_R_CHUNK, _RP, 128), jnp.float32),
            pltpu.VMEM((_R_CHUNK, _RP, 128), jnp.float32),
            pltpu.VMEM((1, 4 * _L), jnp.float32),
            pltpu.SemaphoreType.DMA,
            pltpu.SemaphoreType.DMA,
            pltpu.SemaphoreType.DMA,
        ],
    )
    return f(loc)


def _combine_body(sc_ref, tc_ref, out_ref):
    xy_sum = jnp.sum(sc_ref[:, 0, 0:_L])
    wh_sum = jnp.sum(sc_ref[:, 0, _L:2 * _L])
    pc_sum = jnp.sum(sc_ref[:, 0, 2 * _L:3 * _L])
    nc_sum = jnp.sum(sc_ref[:, 0, 3 * _L:4 * _L])
    class_sum = tc_ref[0]
    npos = tc_ref[1]
    class_loss = class_sum / jnp.maximum(float(_CLASS_NUM) * npos, 1.0)
    xy_loss = xy_sum / jnp.maximum(2.0 * npos, 1.0)
    wh_loss = wh_sum / jnp.maximum(2.0 * npos, 1.0)
    pos_conf = pc_sum / jnp.maximum(npos, 1.0)
    neg_conf = nc_sum / jnp.maximum(float(_BOX_NUM * _N) - npos, 1.0)
    out_ref[0] = (class_loss + 2.0 * pos_conf + 0.5 * neg_conf
                  + 5.0 * xy_loss + 5.0 * wh_loss)
    out_ref[1] = class_loss
    out_ref[2] = xy_loss
    out_ref[3] = wh_loss
    out_ref[4] = pos_conf
    out_ref[5] = neg_conf


def _combine(sc_out, tc_out):
    return pl.pallas_call(
        _combine_body,
        in_specs=[
            pl.BlockSpec(memory_space=pltpu.VMEM),
            pl.BlockSpec(memory_space=pltpu.SMEM),
        ],
        out_specs=pl.BlockSpec(memory_space=pltpu.SMEM),
        out_shape=jax.ShapeDtypeStruct((6,), jnp.float32),
    )(sc_out, tc_out)


@jax.jit
def _yolo_loss(p, gt):
    loc, tc_out = _tc_class(p, gt)
    sc_out = _sc_loc(loc)
    out = _combine(sc_out, tc_out)
    return (out[0], out[1], out[2], out[3], out[4], out[5])


def kernel(p, gt):
    return _yolo_loss(p, gt)
